# baseline (device time: 124390 ns/iter reference)
import jax
import jax.numpy as jnp
from jax import lax
from jax.experimental import pallas as pl
from jax.experimental.pallas import tpu as pltpu

N_DEV = 4
M = 4096
K = 4096
N = 8192
M_BLK = M // N_DEV
K_BLK = K // N_DEV
N_TILE = 512
N_TILES = N // N_TILE

_SLOT_ORDER = (0, 1, 3, 2)
W_DEPTH = 6
OUT_BUFS = 2


def kernel(x, w_mat):
    def body(x_hbm, w_hbm, out_hbm, stage, xbf, xg, wbuf, acc, ostage,
             stage_sems, wsems, send_sems, recv_sems, out_sems):
        me = lax.axis_index("i")

        barrier = pltpu.get_barrier_semaphore()
        for d in range(1, N_DEV):
            pl.semaphore_signal(
                barrier, inc=1,
                device_id=((me + d) % N_DEV,),
                device_id_type=pl.DeviceIdType.MESH,
            )
        pl.semaphore_wait(barrier, N_DEV - 1)

        owners = [(me + 1) % N_DEV, (me + 3) % N_DEV, me, (me + 2) % N_DEV]

        def stage_copy(k):
            return pltpu.make_async_copy(
                x_hbm.at[pl.ds(owners[k] * M_BLK, M_BLK), :],
                stage.at[k % 2],
                stage_sems.at[k % 2],
            )

        def remote_send(d, src):
            t = (me + d) % N_DEV
            return pltpu.make_async_remote_copy(
                src_ref=src,
                dst_ref=xg.at[N_DEV - d],
                send_sem=send_sems.at[d - 1],
                recv_sem=recv_sems.at[N_DEV - d],
                device_id=(t,),
                device_id_type=pl.DeviceIdType.MESH,
            )

        stage_copy(0).start()
        stage_copy(1).start()
        sends = {}
        for k in range(N_DEV):
            stage_copy(k).wait()
            xbf[k] = stage[k % 2].astype(jnp.bfloat16)
            if k + 2 < N_DEV:
                stage_copy(k + 2).start()
            if k == 0:
                sends[1] = remote_send(1, xbf.at[0])
                sends[1].start()
            elif k == 1:
                sends[3] = remote_send(3, xbf.at[1])
                sends[3].start()

        pairs = [(r, j) for r in _SLOT_ORDER for j in range(N_TILES)]

        def w_copy(idx):
            r, j = pairs[idx]
            src_k = (me + r) % N_DEV
            return pltpu.make_async_copy(
                w_hbm.at[pl.ds(src_k * K_BLK, K_BLK),
                         pl.ds(j * N_TILE, N_TILE)],
                wbuf.at[idx % W_DEPTH],
                wsems.at[idx % W_DEPTH],
            )

        def out_copy(j):
            return pltpu.make_async_copy(
                ostage.at[j % OUT_BUFS],
                out_hbm.at[:, pl.ds(j * N_TILE, N_TILE)],
                out_sems.at[j % OUT_BUFS],
            )

        for idx in range(W_DEPTH):
            w_copy(idx).start()

        s2 = None
        for idx, (r, j) in enumerate(pairs):
            hop = idx // N_TILES
            if j == 0 and hop == 1:
                sends[1].wait_send()
                sends[3].wait_send()
                s2 = remote_send(2, xbf.at[3])
                s2.start()
            if j == 0 and r != 0:
                recv = pltpu.make_async_remote_copy(
                    src_ref=xbf.at[0],
                    dst_ref=xg.at[r],
                    send_sem=send_sems.at[0],
                    recv_sem=recv_sems.at[r],
                    device_id=(0,),
                    device_id_type=pl.DeviceIdType.MESH,
                )
                recv.wait_recv()
            w_copy(idx).wait()

            nsl = pl.ds(j * N_TILE, N_TILE)
            lhs = xbf[2] if hop == 0 else xg[r]
            partial = jnp.dot(lhs, wbuf[idx % W_DEPTH].astype(jnp.bfloat16),
                              preferred_element_type=jnp.float32)
            if hop == 0:
                acc[:, nsl] = partial.astype(jnp.bfloat16)
            elif hop < N_DEV - 1:
                acc[:, nsl] = (acc[:, nsl] + partial).astype(jnp.bfloat16)
            else:
                if j >= OUT_BUFS:
                    out_copy(j - OUT_BUFS).wait()
                ostage[j % OUT_BUFS] = jnp.maximum(acc[:, nsl] + partial, 0.0)
                out_copy(j).start()

            nxt = idx + W_DEPTH
            if nxt < len(pairs):
                w_copy(nxt).start()

        for j in range(N_TILES - OUT_BUFS, N_TILES):
            out_copy(j).wait()
        s2.wait_send()

    return pl.pallas_call(
        body,
        out_shape=jax.ShapeDtypeStruct((M_BLK, N), jnp.float32),
        in_specs=[
            pl.BlockSpec(memory_space=pl.ANY),
            pl.BlockSpec(memory_space=pl.ANY),
        ],
        out_specs=pl.BlockSpec(memory_space=pl.ANY),
        scratch_shapes=[
            pltpu.VMEM((2, M_BLK, K_BLK), jnp.float32),
            pltpu.VMEM((N_DEV, M_BLK, K_BLK), jnp.bfloat16),
            pltpu.VMEM((N_DEV, M_BLK, K_BLK), jnp.bfloat16),
            pltpu.VMEM((W_DEPTH, K_BLK, N_TILE), jnp.float32),
            pltpu.VMEM((M_BLK, N), jnp.bfloat16),
            pltpu.VMEM((OUT_BUFS, M_BLK, N_TILE), jnp.float32),
            pltpu.SemaphoreType.DMA((2,)),
            pltpu.SemaphoreType.DMA((W_DEPTH,)),
            pltpu.SemaphoreType.DMA((3,)),
            pltpu.SemaphoreType.DMA((N_DEV,)),
            pltpu.SemaphoreType.DMA((OUT_BUFS,)),
        ],
        compiler_params=pltpu.CompilerParams(
            collective_id=0,
            vmem_limit_bytes=100 * 1024 * 1024,
        ),
    )(x, w_mat)


# device time: 117916 ns/iter; 1.0549x vs baseline; 1.0549x over previous
import jax
import jax.numpy as jnp
from jax import lax
from jax.experimental import pallas as pl
from jax.experimental.pallas import tpu as pltpu

N_DEV = 4
M = 4096
K = 4096
N = 8192
M_BLK = M // N_DEV
K_BLK = K // N_DEV
N_TILE = 1024
N_TILES = N // N_TILE

_SLOT_ORDER = (0, 1, 3, 2)
W_DEPTH = 3
OUT_BUFS = 2


def kernel(x, w_mat):
    def body(x_hbm, w_hbm, out_hbm, stage, xbf, xg, wbuf, acc, ostage,
             stage_sems, wsems, send_sems, recv_sems, out_sems):
        me = lax.axis_index("i")

        barrier = pltpu.get_barrier_semaphore()
        for d in range(1, N_DEV):
            pl.semaphore_signal(
                barrier, inc=1,
                device_id=((me + d) % N_DEV,),
                device_id_type=pl.DeviceIdType.MESH,
            )
        pl.semaphore_wait(barrier, N_DEV - 1)

        owners = [(me + 1) % N_DEV, (me + 3) % N_DEV, me, (me + 2) % N_DEV]

        def stage_copy(k):
            return pltpu.make_async_copy(
                x_hbm.at[pl.ds(owners[k] * M_BLK, M_BLK), :],
                stage.at[k % 2],
                stage_sems.at[k % 2],
            )

        def remote_send(d, src):
            t = (me + d) % N_DEV
            return pltpu.make_async_remote_copy(
                src_ref=src,
                dst_ref=xg.at[N_DEV - d],
                send_sem=send_sems.at[d - 1],
                recv_sem=recv_sems.at[N_DEV - d],
                device_id=(t,),
                device_id_type=pl.DeviceIdType.MESH,
            )

        stage_copy(0).start()
        stage_copy(1).start()
        sends = {}
        for k in range(N_DEV):
            stage_copy(k).wait()
            xbf[k] = stage[k % 2].astype(jnp.bfloat16)
            if k + 2 < N_DEV:
                stage_copy(k + 2).start()
            if k == 0:
                sends[1] = remote_send(1, xbf.at[0])
                sends[1].start()
            elif k == 1:
                sends[3] = remote_send(3, xbf.at[1])
                sends[3].start()

        pairs = [(r, j) for r in _SLOT_ORDER for j in range(N_TILES)]

        def w_copy(idx):
            r, j = pairs[idx]
            src_k = (me + r) % N_DEV
            return pltpu.make_async_copy(
                w_hbm.at[pl.ds(src_k * K_BLK, K_BLK),
                         pl.ds(j * N_TILE, N_TILE)],
                wbuf.at[idx % W_DEPTH],
                wsems.at[idx % W_DEPTH],
            )

        def out_copy(j):
            return pltpu.make_async_copy(
                ostage.at[j % OUT_BUFS],
                out_hbm.at[:, pl.ds(j * N_TILE, N_TILE)],
                out_sems.at[j % OUT_BUFS],
            )

        for idx in range(W_DEPTH):
            w_copy(idx).start()

        s2 = None
        for idx, (r, j) in enumerate(pairs):
            hop = idx // N_TILES
            if j == 0 and hop == 1:
                sends[1].wait_send()
                sends[3].wait_send()
                s2 = remote_send(2, xbf.at[3])
                s2.start()
            if j == 0 and r != 0:
                recv = pltpu.make_async_remote_copy(
                    src_ref=xbf.at[0],
                    dst_ref=xg.at[r],
                    send_sem=send_sems.at[0],
                    recv_sem=recv_sems.at[r],
                    device_id=(0,),
                    device_id_type=pl.DeviceIdType.MESH,
                )
                recv.wait_recv()
            w_copy(idx).wait()

            nsl = pl.ds(j * N_TILE, N_TILE)
            lhs = xbf[2] if hop == 0 else xg[r]
            partial = lax.dot_general(
                lhs, wbuf[idx % W_DEPTH],
                (((1,), (0,)), ((), ())),
                preferred_element_type=jnp.float32,
            )
            if hop == 0:
                acc[:, nsl] = partial.astype(jnp.bfloat16)
            elif hop < N_DEV - 1:
                acc[:, nsl] = (acc[:, nsl] + partial).astype(jnp.bfloat16)
            else:
                if j >= OUT_BUFS:
                    out_copy(j - OUT_BUFS).wait()
                ostage[j % OUT_BUFS] = jnp.maximum(acc[:, nsl] + partial, 0.0)
                out_copy(j).start()

            nxt = idx + W_DEPTH
            if nxt < len(pairs):
                w_copy(nxt).start()

        for j in range(N_TILES - OUT_BUFS, N_TILES):
            out_copy(j).wait()
        s2.wait_send()

    return pl.pallas_call(
        body,
        out_shape=jax.ShapeDtypeStruct((M_BLK, N), jnp.float32),
        in_specs=[
            pl.BlockSpec(memory_space=pl.ANY),
            pl.BlockSpec(memory_space=pl.ANY),
        ],
        out_specs=pl.BlockSpec(memory_space=pl.ANY),
        scratch_shapes=[
            pltpu.VMEM((2, M_BLK, K_BLK), jnp.float32),
            pltpu.VMEM((N_DEV, M_BLK, K_BLK), jnp.bfloat16),
            pltpu.VMEM((N_DEV, M_BLK, K_BLK), jnp.bfloat16),
            pltpu.VMEM((W_DEPTH, K_BLK, N_TILE), jnp.float32),
            pltpu.VMEM((M_BLK, N), jnp.bfloat16),
            pltpu.VMEM((OUT_BUFS, M_BLK, N_TILE), jnp.float32),
            pltpu.SemaphoreType.DMA((2,)),
            pltpu.SemaphoreType.DMA((W_DEPTH,)),
            pltpu.SemaphoreType.DMA((3,)),
            pltpu.SemaphoreType.DMA((N_DEV,)),
            pltpu.SemaphoreType.DMA((OUT_BUFS,)),
        ],
        compiler_params=pltpu.CompilerParams(
            collective_id=0,
            vmem_limit_bytes=100 * 1024 * 1024,
        ),
    )(x, w_mat)
